# Initial kernel scaffold; baseline (speedup 1.0000x reference)
import jax
import jax.numpy as jnp
from jax.experimental import pallas as pl


def kernel(coormeanAngles, labelsAngles, attention):
    raise NotImplementedError("write your pallas kernel here")


# speedup vs baseline: 1.2631x; 1.2631x over previous
import jax
import jax.numpy as jnp
from jax import lax
from jax.experimental import pallas as pl
from jax.experimental.pallas import tpu as pltpu

_N = 4096
_TOTAL = 16384
_S = 128
_BR = 128
_GRID = _N // _BR


def _xor_shuffle(x, j, c_iota, r_iota):
    if j < _S:
        left = jnp.roll(x, -j, axis=1)
        right = jnp.roll(x, j, axis=1)
        islow = (c_iota & j) == 0
    else:
        jr = j // _S
        left = jnp.roll(x, -jr, axis=0)
        right = jnp.roll(x, jr, axis=0)
        islow = (r_iota & jr) == 0
    return jnp.where(islow, left, right), islow


def _bitonic_sort_desc(x):
    c_iota = lax.broadcasted_iota(jnp.int32, (_S, _S), 1)
    r_iota = lax.broadcasted_iota(jnp.int32, (_S, _S), 0)
    x = -x
    k = 2
    while k <= _TOTAL:
        j = k // 2
        while j >= 1:
            partner, islow = _xor_shuffle(x, j, c_iota, r_iota)
            if k >= _TOTAL:
                up = jnp.full((_S, _S), True)
            elif k < _S:
                up = (c_iota & k) == 0
            else:
                up = (r_iota & (k // _S)) == 0
            minv = jnp.minimum(x, partner)
            maxv = jnp.maximum(x, partner)
            x = jnp.where(up == islow, minv, maxv)
            j //= 2
        k *= 2
    return -x


def _body(coor_ref, lab_ref, att_ref, out_ref, sorted_ref):
    g = pl.program_id(0)

    @pl.when(g == 0)
    def _():
        sorted_ref[...] = _bitonic_sort_desc(att_ref[...])
        out_ref[...] = jnp.zeros_like(out_ref)

    s = jnp.sum(jnp.abs(coor_ref[...] - lab_ref[...]), axis=1, keepdims=True)
    w = sorted_ref[pl.ds(g, 1), :]
    out_ref[...] += jnp.dot(w, s, preferred_element_type=jnp.float32)


def kernel(coormeanAngles, labelsAngles, attention):
    att2d = attention.reshape(_S, _S)
    out = pl.pallas_call(
        _body,
        grid=(_GRID,),
        in_specs=[
            pl.BlockSpec((_BR, _N), lambda g: (g, 0)),
            pl.BlockSpec((_BR, _N), lambda g: (g, 0)),
            pl.BlockSpec((_S, _S), lambda g: (0, 0)),
        ],
        out_specs=pl.BlockSpec((1, 1), lambda g: (0, 0)),
        out_shape=jax.ShapeDtypeStruct((1, 1), jnp.float32),
        scratch_shapes=[pltpu.VMEM((_S, _S), jnp.float32)],
    )(coormeanAngles, labelsAngles, att2d)
    return out[0, 0]


# speedup vs baseline: 1.3158x; 1.0417x over previous
import jax
import jax.numpy as jnp
from jax import lax
from jax.experimental import pallas as pl
from jax.experimental.pallas import tpu as pltpu

_N = 4096
_TOTAL = 16384
_S = 128
_BR = 128
_GRID = _N // _BR
_STAGES_PER_STEP = 4


def _stage_list():
    stages = []
    k = 2
    while k <= _TOTAL:
        j = k // 2
        while j >= 1:
            stages.append((k, j))
            j //= 2
        k *= 2
    return stages


_STAGES = _stage_list()


def _apply_stage(x, k, j, c_iota, r_iota):
    if j < _S:
        left = jnp.roll(x, -j, axis=1)
        right = jnp.roll(x, j, axis=1)
        islow = (c_iota & j) == 0
    else:
        jr = j // _S
        left = jnp.roll(x, -jr, axis=0)
        right = jnp.roll(x, jr, axis=0)
        islow = (r_iota & jr) == 0
    partner = jnp.where(islow, left, right)
    if k >= _TOTAL:
        up = jnp.full((_S, _S), True)
    elif k < _S:
        up = (c_iota & k) == 0
    else:
        up = (r_iota & (k // _S)) == 0
    minv = jnp.minimum(x, partner)
    maxv = jnp.maximum(x, partner)
    return jnp.where(up == islow, minv, maxv)


def _body(coor_ref, lab_ref, att_ref, out_ref, work_ref, sums_ref):
    g = pl.program_id(0)
    c_iota = lax.broadcasted_iota(jnp.int32, (_S, _S), 1)
    r_iota = lax.broadcasted_iota(jnp.int32, (_S, _S), 0)

    @pl.when(g == 0)
    def _():
        work_ref[...] = -att_ref[...]
        sums_ref[...] = jnp.zeros_like(sums_ref)
        out_ref[...] = jnp.zeros_like(out_ref)

    n_chunks = -(-len(_STAGES) // _STAGES_PER_STEP)
    for c in range(n_chunks):
        chunk = _STAGES[c * _STAGES_PER_STEP:(c + 1) * _STAGES_PER_STEP]

        @pl.when(g == c)
        def _(chunk=chunk):
            x = work_ref[...]
            for (k, j) in chunk:
                x = _apply_stage(x, k, j, c_iota, r_iota)
            work_ref[...] = x

    s = jnp.sum(jnp.abs(coor_ref[...] - lab_ref[...]), axis=1, keepdims=True)
    onehot = (c_iota[0:1, :] == g).astype(jnp.float32)
    sums_ref[...] += jnp.dot(s, onehot, preferred_element_type=jnp.float32)

    @pl.when(g == _GRID - 1)
    def _():
        sorted_desc = -work_ref[...]
        out_ref[...] += jnp.sum(
            sorted_desc * sums_ref[...].T, dtype=jnp.float32
        ).reshape(1, 1)


def kernel(coormeanAngles, labelsAngles, attention):
    att2d = attention.reshape(_S, _S)
    out = pl.pallas_call(
        _body,
        grid=(_GRID,),
        in_specs=[
            pl.BlockSpec((_BR, _N), lambda g: (g, 0)),
            pl.BlockSpec((_BR, _N), lambda g: (g, 0)),
            pl.BlockSpec((_S, _S), lambda g: (0, 0)),
        ],
        out_specs=pl.BlockSpec((1, 1), lambda g: (0, 0)),
        out_shape=jax.ShapeDtypeStruct((1, 1), jnp.float32),
        scratch_shapes=[
            pltpu.VMEM((_S, _S), jnp.float32),
            pltpu.VMEM((_S, _S), jnp.float32),
        ],
    )(coormeanAngles, labelsAngles, att2d)
    return out[0, 0]


# speedup vs baseline: 1.5974x; 1.2140x over previous
import jax
import jax.numpy as jnp
from jax import lax
from jax.experimental import pallas as pl
from jax.experimental.pallas import tpu as pltpu

_N = 4096
_TOTAL = 16384
_S = 128
_BR = 256
_GRID = _N // _BR
_STAGES_PER_STEP = 8


def _stage_list():
    stages = []
    k = 2
    while k <= _TOTAL:
        j = k // 2
        while j >= 1:
            stages.append((k, j))
            j //= 2
        k *= 2
    return stages


_STAGES = _stage_list()


def _apply_stage(x, k, j, c_iota, r_iota):
    if j < _S:
        left = jnp.roll(x, -j, axis=1)
        right = jnp.roll(x, j, axis=1)
        islow = (c_iota & j) == 0
    else:
        jr = j // _S
        left = jnp.roll(x, -jr, axis=0)
        right = jnp.roll(x, jr, axis=0)
        islow = (r_iota & jr) == 0
    partner = jnp.where(islow, left, right)
    if k >= _TOTAL:
        up = jnp.full((_S, _S), True)
    elif k < _S:
        up = (c_iota & k) == 0
    else:
        up = (r_iota & (k // _S)) == 0
    minv = jnp.minimum(x, partner)
    maxv = jnp.maximum(x, partner)
    return jnp.where(up == islow, minv, maxv)


def _body(coor_ref, lab_ref, att_ref, out_ref, work_ref, sums_ref):
    g = pl.program_id(0)
    c_iota = lax.broadcasted_iota(jnp.int32, (_S, _S), 1)
    r_iota = lax.broadcasted_iota(jnp.int32, (_S, _S), 0)

    @pl.when(g == 0)
    def _():
        work_ref[...] = -att_ref[...]
        sums_ref[...] = jnp.zeros_like(sums_ref)
        out_ref[...] = jnp.zeros_like(out_ref)

    n_chunks = -(-len(_STAGES) // _STAGES_PER_STEP)
    for c in range(n_chunks):
        chunk = _STAGES[c * _STAGES_PER_STEP:(c + 1) * _STAGES_PER_STEP]

        @pl.when(g == c)
        def _(chunk=chunk):
            x = work_ref[...]
            for (k, j) in chunk:
                x = _apply_stage(x, k, j, c_iota, r_iota)
            work_ref[...] = x

    s = jnp.sum(jnp.abs(coor_ref[...] - lab_ref[...]), axis=1, keepdims=True)
    nsub = _BR // _S
    for h in range(nsub):
        onehot = (c_iota[0:1, :] == g * nsub + h).astype(jnp.float32)
        sums_ref[...] += jnp.dot(
            s[_S * h:_S * (h + 1)], onehot, preferred_element_type=jnp.float32
        )

    @pl.when(g == _GRID - 1)
    def _():
        sorted_desc = -work_ref[...]
        out_ref[...] += jnp.sum(
            sorted_desc * sums_ref[...].T, dtype=jnp.float32
        ).reshape(1, 1)


def kernel(coormeanAngles, labelsAngles, attention):
    att2d = attention.reshape(_S, _S)
    out = pl.pallas_call(
        _body,
        grid=(_GRID,),
        in_specs=[
            pl.BlockSpec((_BR, _N), lambda g: (g, 0)),
            pl.BlockSpec((_BR, _N), lambda g: (g, 0)),
            pl.BlockSpec((_S, _S), lambda g: (0, 0)),
        ],
        out_specs=pl.BlockSpec((1, 1), lambda g: (0, 0)),
        out_shape=jax.ShapeDtypeStruct((1, 1), jnp.float32),
        scratch_shapes=[
            pltpu.VMEM((_S, _S), jnp.float32),
            pltpu.VMEM((_S, _S), jnp.float32),
        ],
    )(coormeanAngles, labelsAngles, att2d)
    return out[0, 0]
